# baseline (speedup 1.0000x reference)
import functools

import jax
import jax.numpy as jnp
from jax import lax
from jax.experimental import pallas as pl
from jax.experimental.pallas import tpu as pltpu
from jax.experimental.pallas import tpu_sc as plsc

D = 64
B = 819200
NC = 2
NS = 16
NW = NC * NS
BPW = B // NW
SUB = 128
CHUNK = SUB
NCH = BPW // CHUNK
L = 16


def _sc_gather(idx3, pidx3, table128):
    mesh = plsc.VectorSubcoreMesh(core_axis_name="c", subcore_axis_name="s")

    @functools.partial(
        pl.kernel,
        mesh=mesh,
        compiler_params=pltpu.CompilerParams(needs_layout_passes=False),
        out_type=jax.ShapeDtypeStruct((B, D), jnp.float32),
        scratch_types=[
            pltpu.VMEM((NCH, SUB), jnp.int32),
            pltpu.VMEM((NCH, SUB), jnp.int32),
            pltpu.VMEM((CHUNK, 2 * D), jnp.float32),
            pltpu.VMEM((CHUNK, 2 * D), jnp.float32),
            pltpu.VMEM((CHUNK, D), jnp.float32),
            pltpu.VMEM((CHUNK, D), jnp.float32),
            pltpu.SemaphoreType.DMA,
            pltpu.SemaphoreType.DMA,
            pltpu.SemaphoreType.DMA,
            pltpu.SemaphoreType.DMA,
        ],
    )
    def k(idx_hbm, pidx_hbm, table_hbm, out_hbm, idx_v, pidx_v,
          pair0, pair1, outb0, outb1, gsem0, gsem1, wsem0, wsem1):
        wid = lax.axis_index("s") * NC + lax.axis_index("c")
        pltpu.sync_copy(idx_hbm.at[wid], idx_v)
        pltpu.sync_copy(pidx_hbm.at[wid], pidx_v)
        base = wid * BPW
        pairs = (pair0, pair1)
        outbs = (outb0, outb1)
        gsems = (gsem0, gsem1)
        wsems = (wsem0, wsem1)

        def issue_gather(c, b):
            pltpu.async_copy(table_hbm.at[pidx_v.at[c]], pairs[b], gsems[b])

        def wait_gather(c, b):
            pltpu.make_async_copy(
                table_hbm.at[pidx_v.at[c]], pairs[b], gsems[b]).wait()

        def select(c, b):
            for g in range(CHUNK // L):
                rowv = lax.iota(jnp.int32, L) + (g * L)
                idxv = idx_v[c, pl.ds(g * L, L)]
                bitv = (idxv & 1) * D
                for cc in range(D):
                    colv = bitv + cc
                    vals = plsc.load_gather(pairs[b], [rowv, colv])
                    ccv = jnp.full((L,), cc, jnp.int32)
                    plsc.store_scatter(outbs[b], [rowv, ccv], vals)

        def issue_write(c, b):
            pltpu.async_copy(
                outbs[b], out_hbm.at[pl.ds(base + c * CHUNK, CHUNK)],
                wsems[b])

        def wait_write(c, b):
            pltpu.make_async_copy(
                outbs[b], out_hbm.at[pl.ds(base + c * CHUNK, CHUNK)],
                wsems[b]).wait()

        issue_gather(0, 0)
        issue_gather(1, 1)

        def body(p, carry):
            for par in range(2):
                c = 2 * p + par
                b = par
                wait_gather(c, b)
                pl.when(p >= 1)(lambda: wait_write(c - 2, b))
                select(c, b)
                issue_write(c, b)
                pl.when(p < NCH // 2 - 1)(lambda: issue_gather(c + 2, b))
            return carry

        lax.fori_loop(0, NCH // 2, body, 0)
        wait_write(NCH - 2, 0)
        wait_write(NCH - 1, 1)

    return k(idx3, pidx3, table128)


def kernel(indices, batch_sizes, table):
    del batch_sizes
    idx = indices.astype(jnp.int32)
    idx3 = idx.reshape(NW, NCH, SUB)
    pidx3 = lax.shift_right_logical(idx, 1).reshape(NW, NCH, SUB)
    table128 = table.reshape(500000, 2 * D)
    return _sc_gather(idx3, pidx3, table128)


# speedup vs baseline: 2.2304x; 2.2304x over previous
import functools

import jax
import jax.numpy as jnp
from jax import lax
from jax.experimental import pallas as pl
from jax.experimental.pallas import tpu as pltpu
from jax.experimental.pallas import tpu_sc as plsc

D = 64
B = 819200
NC = 2
NS = 16
NW = NC * NS
BPW = B // NW
SUB = 128
GPC = 4
CHUNK = SUB * GPC
NSUB = BPW // SUB
NCH = NSUB // GPC


def _sc_gather(idx3, table):
    mesh = plsc.VectorSubcoreMesh(core_axis_name="c", subcore_axis_name="s")

    @functools.partial(
        pl.kernel,
        mesh=mesh,
        compiler_params=pltpu.CompilerParams(use_tc_tiling_on_sc=False),
        out_type=jax.ShapeDtypeStruct((B // CHUNK, CHUNK, D), jnp.float32),
        scratch_types=[
            pltpu.VMEM((NSUB, SUB), jnp.int32),
            pltpu.VMEM((CHUNK, D), jnp.float32),
            pltpu.VMEM((CHUNK, D), jnp.float32),
            pltpu.SemaphoreType.DMA,
            pltpu.SemaphoreType.DMA,
            pltpu.SemaphoreType.DMA,
            pltpu.SemaphoreType.DMA,
        ],
    )
    def k(idx_hbm, table_hbm, out_hbm, idx_v, buf0, buf1,
          gsem0, gsem1, wsem0, wsem1):
        wid = lax.axis_index("s") * NC + lax.axis_index("c")
        pltpu.sync_copy(idx_hbm.at[wid], idx_v)
        base = wid * BPW
        bufs = (buf0, buf1)
        gsems = (gsem0, gsem1)
        wsems = (wsem0, wsem1)

        def issue_gather(c, b):
            for i in range(GPC):
                pltpu.async_copy(
                    table_hbm.at[idx_v.at[c * GPC + i]],
                    bufs[b].at[pl.ds(i * SUB, SUB)], gsems[b])

        def wait_gather(c, b):
            for i in range(GPC):
                pltpu.make_async_copy(
                    table_hbm.at[idx_v.at[c * GPC + i]],
                    bufs[b].at[pl.ds(i * SUB, SUB)], gsems[b]).wait()

        def issue_write(c, b):
            pltpu.async_copy(
                bufs[b],
                out_hbm.at[wid * NCH + c],
                wsems[b])

        def wait_write(c, b):
            pltpu.make_async_copy(
                bufs[b],
                out_hbm.at[wid * NCH + c],
                wsems[b]).wait()

        issue_gather(0, 0)
        wait_gather(0, 0)
        issue_write(0, 0)
        issue_gather(1, 1)

        def body(p, carry):
            c = 1 + 2 * p
            wait_gather(c, 1)
            issue_write(c, 1)
            wait_write(c - 1, 0)
            issue_gather(c + 1, 0)
            wait_gather(c + 1, 0)
            issue_write(c + 1, 0)
            wait_write(c, 1)
            issue_gather(c + 2, 1)
            return carry

        lax.fori_loop(0, (NCH - 2) // 2, body, 0)
        wait_gather(NCH - 1, 1)
        issue_write(NCH - 1, 1)
        wait_write(NCH - 2, 0)
        wait_write(NCH - 1, 1)

    return k(idx3, table)


def kernel(indices, batch_sizes, table):
    del batch_sizes
    idx3 = indices.astype(jnp.int32).reshape(NW, NSUB, SUB)
    return _sc_gather(idx3, table).reshape(B, D)
